# baseline (speedup 1.0000x reference)
import functools

import jax
import jax.numpy as jnp
from jax import lax
from jax.experimental import pallas as pl
from jax.experimental.pallas import tpu as pltpu
from jax.experimental.pallas import tpu_sc as plsc

ROWS = 1024
SEGS = 50
VOCAB = 1000
NC, NS, L = 2, 16, 16
NW = NC * NS
ROWS_PER_W = ROWS // NW

_SEG_STARTS = (0, 16, 32, 34)
_ZERO_STARTS = tuple(range(0, VOCAB - L + 1, L)) + (VOCAB - L,)


def _onehot_body(x_hbm, out_hbm, buf, idx_v):
    wid = lax.axis_index("s") * NC + lax.axis_index("c")
    base = wid * ROWS_PER_W

    iota = lax.iota(jnp.int32, L)
    zeros = jnp.zeros((L,), jnp.float32)
    ones = jnp.ones((L,), jnp.float32)

    pltpu.sync_copy(x_hbm.at[pl.ds(base, ROWS_PER_W), :], idx_v)

    def zero_row(r, _):
        for s in _ZERO_STARTS:
            buf[r, pl.ds(s, L)] = zeros
        return 0

    lax.fori_loop(0, SEGS, zero_row, 0)

    def do_row(c, _):
        for s in _SEG_STARTS:
            cols = idx_v[c, pl.ds(s, L)]
            plsc.store_scatter(buf, [s + iota, cols], ones)
        pltpu.sync_copy(buf, out_hbm.at[base + c])
        for s in _SEG_STARTS:
            cols = idx_v[c, pl.ds(s, L)]
            plsc.store_scatter(buf, [s + iota, cols], zeros)
        return 0

    lax.fori_loop(0, ROWS_PER_W, do_row, 0)


@functools.partial(jax.jit, static_argnums=())
def _onehot_sc(x):
    mesh = plsc.VectorSubcoreMesh(core_axis_name="c", subcore_axis_name="s")
    return pl.kernel(
        _onehot_body,
        out_type=jax.ShapeDtypeStruct((ROWS, SEGS, VOCAB), jnp.float32),
        mesh=mesh,
        scratch_types=[
            pltpu.VMEM((SEGS, VOCAB), jnp.float32),
            pltpu.VMEM((ROWS_PER_W, SEGS), jnp.int32),
        ],
        compiler_params=pltpu.CompilerParams(needs_layout_passes=False),
    )(x)


def kernel(x):
    return _onehot_sc(x.astype(jnp.int32))
